# Initial kernel scaffold; baseline (speedup 1.0000x reference)
import jax
import jax.numpy as jnp
from jax.experimental import pallas as pl


def kernel(x, edge_index, edge_attr, batch, We1, be1, W1a, b1a, g1, bt1, W1b, b1b, We2, be2, W2a, b2a, g2, bt2, W2b, b2b, We3, be3, W3a, b3a, g3, bt3, W3b, b3b, Wl1, bl1, Wl2, bl2):
    raise NotImplementedError("write your pallas kernel here")


# speedup vs baseline: 2.3620x; 2.3620x over previous
import functools

import jax
import jax.numpy as jnp
from jax import lax
from jax.experimental import pallas as pl
from jax.experimental.pallas import tpu as pltpu
from jax.experimental.pallas import tpu_sc as plsc

NC = 2
NS = 16
NW = NC * NS
W = 128


def _edge_feats_body(ea_ref, We1_ref, be1_ref, We2_ref, be2_ref, We3_ref,
                     be3_ref, e1_ref, e2_ref, e3_ref):
    ea = ea_ref[...]
    e1_ref[...] = jnp.dot(ea, We1_ref[...],
                          preferred_element_type=jnp.float32) + be1_ref[...]
    e2_ref[...] = jnp.dot(ea, We2_ref[...],
                          preferred_element_type=jnp.float32) + be2_ref[...]
    e3_ref[...] = jnp.dot(ea, We3_ref[...],
                          preferred_element_type=jnp.float32) + be3_ref[...]


def _edge_feats(ea, We1, be1, We2, be2, We3, be3):
    E, DE = ea.shape
    BE = 2000
    wspec = pl.BlockSpec((DE, W), lambda i: (0, 0))
    bspec = pl.BlockSpec((1, W), lambda i: (0, 0))
    espec = pl.BlockSpec((BE, W), lambda i: (i, 0))
    eshape = jax.ShapeDtypeStruct((E, W), jnp.float32)
    return pl.pallas_call(
        _edge_feats_body,
        grid=(E // BE,),
        in_specs=[pl.BlockSpec((BE, DE), lambda i: (i, 0)),
                  wspec, bspec, wspec, bspec, wspec, bspec],
        out_specs=[espec, espec, espec],
        out_shape=[eshape, eshape, eshape],
    )(ea, We1, be1, We2, be2, We3, be3)


def _sc_aggregate(h, src, dst, e):
    N, D = h.shape
    E = src.shape[0]
    C = 80
    EPW = E // NW
    NCHUNK = EPW // C
    ACT = 10
    ROWS = N // ACT
    RZ = 200
    mesh = plsc.VectorSubcoreMesh(core_axis_name="c", subcore_axis_name="s")

    @functools.partial(
        pl.kernel,
        out_type=jax.ShapeDtypeStruct((NC, N, D), jnp.float32),
        mesh=mesh,
        scratch_types=[
            pltpu.VMEM((C,), jnp.int32),
            pltpu.VMEM((C,), jnp.int32),
            pltpu.VMEM((C, D), jnp.float32),
            pltpu.VMEM((C, D), jnp.float32),
            pltpu.VMEM((RZ, D), jnp.float32),
            pltpu.VMEM_SHARED((N, D), jnp.float32),
            pltpu.SemaphoreType.DMA,
        ],
    )
    def agg(h_hbm, src_hbm, dst_hbm, e_hbm, out_hbm,
            src_v, dst_v, g_v, e_v, z_v, aggr_sh, sem):
        cid = lax.axis_index("c")
        sid = lax.axis_index("s")
        base = (cid * NS + sid) * EPW

        @plsc.parallel_loop(0, RZ)
        def _(i):
            for j in range(D // 16):
                z_v[i, pl.ds(j * 16, 16)] = jnp.zeros((16,), jnp.float32)

        @pl.when(sid < ACT)
        def _():
            for k in range(ROWS // RZ):
                pltpu.sync_copy(z_v, aggr_sh.at[pl.ds(sid * ROWS + k * RZ, RZ)])
        plsc.subcore_barrier()

        def chunk(c, carry):
            off = base + c * C
            pltpu.sync_copy(src_hbm.at[pl.ds(off, C)], src_v)
            pltpu.sync_copy(dst_hbm.at[pl.ds(off, C)], dst_v)
            pltpu.sync_copy(e_hbm.at[pl.ds(off, C)], e_v)
            pltpu.async_copy(h_hbm.at[src_v], g_v, sem).wait()

            @plsc.parallel_loop(0, C)
            def _(i):
                for j in range(D // 16):
                    s = pl.ds(j * 16, 16)
                    g_v[i, s] = jnp.maximum(g_v[i, s] + e_v[i, s], 0.0)

            pltpu.sync_copy(g_v, aggr_sh.at[dst_v], add=True)
            return carry

        lax.fori_loop(0, NCHUNK, chunk, 0)
        plsc.subcore_barrier()

        @pl.when(sid < ACT)
        def _():
            pltpu.sync_copy(aggr_sh.at[pl.ds(sid * ROWS, ROWS)],
                            out_hbm.at[cid, pl.ds(sid * ROWS, ROWS)])

    return agg(h, src, dst, e)


def _mlp_bn(h, Wa_ref, ba_ref, g_ref, bt_ref, Wb_ref, bb_ref):
    t = jnp.dot(h, Wa_ref[...], preferred_element_type=jnp.float32) + ba_ref[...]
    mu = jnp.mean(t, axis=0, keepdims=True)
    var = jnp.mean((t - mu) ** 2, axis=0, keepdims=True)
    t = (t - mu) * lax.rsqrt(var + 1e-5) * g_ref[...] + bt_ref[...]
    t = jnp.maximum(t, 0.0)
    t = jnp.dot(t, Wb_ref[...], preferred_element_type=jnp.float32) + bb_ref[...]
    return jnp.maximum(t, 0.0)


def _dense_body(x_ref, p_ref, Wa_ref, ba_ref, g_ref, bt_ref, Wb_ref, bb_ref,
                o_ref):
    h = x_ref[...] + p_ref[0] + p_ref[1]
    o_ref[...] = _mlp_bn(h, Wa_ref, ba_ref, g_ref, bt_ref, Wb_ref, bb_ref)


def _dense(x, p, Wa, ba, g, bt, Wb, bb):
    N = x.shape[0]
    return pl.pallas_call(
        _dense_body,
        out_shape=jax.ShapeDtypeStruct((N, Wb.shape[1]), jnp.float32),
    )(x, p, Wa, ba, g, bt, Wb, bb)


def _dense_final_body(x_ref, p_ref, Wa_ref, ba_ref, g_ref, bt_ref, Wb_ref,
                      bb_ref, Wl1_ref, bl1_ref, Wl2_ref, bl2_ref, o_ref):
    h = x_ref[...] + p_ref[0] + p_ref[1]
    t = _mlp_bn(h, Wa_ref, ba_ref, g_ref, bt_ref, Wb_ref, bb_ref)
    t = jnp.maximum(
        jnp.dot(t, Wl1_ref[...], preferred_element_type=jnp.float32)
        + bl1_ref[...], 0.0)
    o_ref[...] = jnp.dot(t, Wl2_ref[...],
                         preferred_element_type=jnp.float32) + bl2_ref[...]


def _dense_final(x, p, Wa, ba, g, bt, Wb, bb, Wl1, bl1, Wl2, bl2):
    N = x.shape[0]
    return pl.pallas_call(
        _dense_final_body,
        out_shape=jax.ShapeDtypeStruct((N, Wl2.shape[1]), jnp.float32),
    )(x, p, Wa, ba, g, bt, Wb, bb, Wl1, bl1, Wl2, bl2)


def _pad_cols(a, w=W):
    return jnp.pad(a, ((0, 0), (0, w - a.shape[1])))


def kernel(x, edge_index, edge_attr, batch, We1, be1, W1a, b1a, g1, bt1, W1b,
           b1b, We2, be2, W2a, b2a, g2, bt2, W2b, b2b, We3, be3, W3a, b3a, g3,
           bt3, W3b, b3b, Wl1, bl1, Wl2, bl2):
    src = edge_index[0]
    dst = edge_index[1]
    H = W1a.shape[1]
    We2p, We3p = _pad_cols(We2), _pad_cols(We3)
    be2p = jnp.pad(be2, (0, W - H))[None]
    be3p = jnp.pad(be3, (0, W - H))[None]
    W1bp, W2bp = _pad_cols(W1b), _pad_cols(W2b)
    b1bp = jnp.pad(b1b, (0, W - H))[None]
    b2bp = jnp.pad(b2b, (0, W - H))[None]
    W2ap = jnp.pad(W2a, ((0, W - H), (0, 0)))
    W3ap = jnp.pad(W3a, ((0, W - H), (0, 0)))

    e1, e2, e3 = _edge_feats(edge_attr, We1, be1[None], We2p, be2p, We3p, be3p)
    p1 = _sc_aggregate(x, src, dst, e1)
    h1 = _dense(x, p1, W1a, b1a[None], g1[None], bt1[None], W1bp, b1bp)
    p2 = _sc_aggregate(h1, src, dst, e2)
    h2 = _dense(h1, p2, W2ap, b2a[None], g2[None], bt2[None], W2bp, b2bp)
    p3 = _sc_aggregate(h2, src, dst, e3)
    return _dense_final(h2, p3, W3ap, b3a[None], g3[None], bt3[None], W3b,
                        b3b[None], Wl1, bl1[None], Wl2, bl2[None])


# speedup vs baseline: 4.6001x; 1.9475x over previous
import functools

import jax
import jax.numpy as jnp
from jax import lax
from jax.experimental import pallas as pl
from jax.experimental.pallas import tpu as pltpu
from jax.experimental.pallas import tpu_sc as plsc

NC = 2
NS = 16
NW = NC * NS
W = 128


def _edge_feats_body(ea_ref, We1_ref, be1_ref, We2_ref, be2_ref, We3_ref,
                     be3_ref, e1_ref, e2_ref, e3_ref):
    ea = ea_ref[...]
    e1_ref[...] = jnp.dot(ea, We1_ref[...],
                          preferred_element_type=jnp.float32) + be1_ref[...]
    e2_ref[...] = jnp.dot(ea, We2_ref[...],
                          preferred_element_type=jnp.float32) + be2_ref[...]
    e3_ref[...] = jnp.dot(ea, We3_ref[...],
                          preferred_element_type=jnp.float32) + be3_ref[...]


def _edge_feats(ea, We1, be1, We2, be2, We3, be3):
    E, DE = ea.shape
    BE = 2000
    wspec = pl.BlockSpec((DE, W), lambda i: (0, 0))
    bspec = pl.BlockSpec((1, W), lambda i: (0, 0))
    espec = pl.BlockSpec((BE, W), lambda i: (i, 0))
    eshape = jax.ShapeDtypeStruct((E, W), jnp.float32)
    return pl.pallas_call(
        _edge_feats_body,
        grid=(E // BE,),
        in_specs=[pl.BlockSpec((BE, DE), lambda i: (i, 0)),
                  wspec, bspec, wspec, bspec, wspec, bspec],
        out_specs=[espec, espec, espec],
        out_shape=[eshape, eshape, eshape],
    )(ea, We1, be1, We2, be2, We3, be3)


def _sc_aggregate(h, src, dst, e):
    N, D = h.shape
    E = src.shape[0]
    C = 40
    EPW = E // NW
    NTOT = EPW // C
    NB = 3
    NITER = (NTOT + NB) // NB
    ACT = 10
    ROWS = N // ACT
    RZ = 40
    mesh = plsc.VectorSubcoreMesh(core_axis_name="c", subcore_axis_name="s")

    @functools.partial(
        pl.kernel,
        out_type=jax.ShapeDtypeStruct((NC, N, D), jnp.float32),
        mesh=mesh,
        scratch_types=[
            [pltpu.VMEM((C,), jnp.int32)] * NB,
            [pltpu.VMEM((C,), jnp.int32)] * NB,
            [pltpu.VMEM((C, D), jnp.float32)] * NB,
            [pltpu.VMEM((C, D), jnp.float32)] * NB,
            pltpu.VMEM((RZ, D), jnp.float32),
            pltpu.VMEM_SHARED((N, D), jnp.float32),
            [pltpu.SemaphoreType.DMA] * NB,
            [pltpu.SemaphoreType.DMA] * NB,
            [pltpu.SemaphoreType.DMA] * NB,
        ],
    )
    def agg(h_hbm, src_hbm, dst_hbm, e_hbm, out_hbm,
            src_v, dst_v, g_v, e_v, z_v, aggr_sh, semA, semG, semS):
        cid = lax.axis_index("c")
        sid = lax.axis_index("s")
        base = (cid * NS + sid) * EPW

        def fetch(c, b):
            off = base + c * C
            pltpu.async_copy(src_hbm.at[pl.ds(off, C)], src_v[b], semA[b])
            pltpu.async_copy(dst_hbm.at[pl.ds(off, C)], dst_v[b], semA[b])
            pltpu.async_copy(e_hbm.at[pl.ds(off, C)], e_v[b], semA[b])

        def waitA(b):
            pltpu.make_async_copy(src_hbm.at[pl.ds(0, C)], src_v[b], semA[b]).wait()
            pltpu.make_async_copy(dst_hbm.at[pl.ds(0, C)], dst_v[b], semA[b]).wait()
            pltpu.make_async_copy(e_hbm.at[pl.ds(0, C)], e_v[b], semA[b]).wait()

        def gather(b):
            pltpu.async_copy(h_hbm.at[src_v[b]], g_v[b], semG[b])

        def waitG(b):
            pltpu.make_async_copy(h_hbm.at[src_v[b]], g_v[b], semG[b]).wait()

        def scatter(b):
            pltpu.async_copy(g_v[b], aggr_sh.at[dst_v[b]], semS[b], add=True)

        def waitS(b):
            pltpu.make_async_copy(g_v[b], aggr_sh.at[dst_v[b]], semS[b]).wait()

        @plsc.parallel_loop(0, RZ)
        def _(i):
            for j in range(D // 16):
                z_v[i, pl.ds(j * 16, 16)] = jnp.zeros((16,), jnp.float32)

        @pl.when(sid < ACT)
        def _():
            for k in range(ROWS // RZ):
                pltpu.sync_copy(z_v, aggr_sh.at[pl.ds(sid * ROWS + k * RZ, RZ)])
        plsc.subcore_barrier()

        fetch(0, 0)
        fetch(1, 1)
        waitA(0)
        gather(0)

        def body(k, carry):
            for b in range(NB):
                c = k * NB + b
                nb = (b + 1) % NB

                @pl.when(c + 2 < NTOT)
                def _():
                    fetch(c + 2, (b + 2) % NB)

                @pl.when(c + 1 < NTOT)
                def _():
                    waitA(nb)

                    @pl.when(c >= 2)
                    def _():
                        waitS(nb)
                    gather(nb)

                @pl.when(c < NTOT)
                def _():
                    waitG(b)

                    @plsc.parallel_loop(0, C)
                    def _(i):
                        for j in range(D // 16):
                            s = pl.ds(j * 16, 16)
                            g_v[b][i, s] = jnp.maximum(
                                g_v[b][i, s] + e_v[b][i, s], 0.0)
                    scatter(b)
            return carry

        lax.fori_loop(0, NITER, body, 0)
        for b in range(NB):
            waitS(b)
        plsc.subcore_barrier()

        @pl.when(sid < ACT)
        def _():
            pltpu.sync_copy(aggr_sh.at[pl.ds(sid * ROWS, ROWS)],
                            out_hbm.at[cid, pl.ds(sid * ROWS, ROWS)])

    return agg(h, src, dst, e)


def _mlp_bn(h, Wa_ref, ba_ref, g_ref, bt_ref, Wb_ref, bb_ref):
    t = jnp.dot(h, Wa_ref[...], preferred_element_type=jnp.float32) + ba_ref[...]
    mu = jnp.mean(t, axis=0, keepdims=True)
    var = jnp.mean((t - mu) ** 2, axis=0, keepdims=True)
    t = (t - mu) * lax.rsqrt(var + 1e-5) * g_ref[...] + bt_ref[...]
    t = jnp.maximum(t, 0.0)
    t = jnp.dot(t, Wb_ref[...], preferred_element_type=jnp.float32) + bb_ref[...]
    return jnp.maximum(t, 0.0)


def _dense_body(x_ref, p_ref, Wa_ref, ba_ref, g_ref, bt_ref, Wb_ref, bb_ref,
                o_ref):
    h = x_ref[...] + p_ref[0] + p_ref[1]
    o_ref[...] = _mlp_bn(h, Wa_ref, ba_ref, g_ref, bt_ref, Wb_ref, bb_ref)


def _dense(x, p, Wa, ba, g, bt, Wb, bb):
    N = x.shape[0]
    return pl.pallas_call(
        _dense_body,
        out_shape=jax.ShapeDtypeStruct((N, Wb.shape[1]), jnp.float32),
    )(x, p, Wa, ba, g, bt, Wb, bb)


def _dense_final_body(x_ref, p_ref, Wa_ref, ba_ref, g_ref, bt_ref, Wb_ref,
                      bb_ref, Wl1_ref, bl1_ref, Wl2_ref, bl2_ref, o_ref):
    h = x_ref[...] + p_ref[0] + p_ref[1]
    t = _mlp_bn(h, Wa_ref, ba_ref, g_ref, bt_ref, Wb_ref, bb_ref)
    t = jnp.maximum(
        jnp.dot(t, Wl1_ref[...], preferred_element_type=jnp.float32)
        + bl1_ref[...], 0.0)
    o_ref[...] = jnp.dot(t, Wl2_ref[...],
                         preferred_element_type=jnp.float32) + bl2_ref[...]


def _dense_final(x, p, Wa, ba, g, bt, Wb, bb, Wl1, bl1, Wl2, bl2):
    N = x.shape[0]
    return pl.pallas_call(
        _dense_final_body,
        out_shape=jax.ShapeDtypeStruct((N, Wl2.shape[1]), jnp.float32),
    )(x, p, Wa, ba, g, bt, Wb, bb, Wl1, bl1, Wl2, bl2)


def _pad_cols(a, w=W):
    return jnp.pad(a, ((0, 0), (0, w - a.shape[1])))


def kernel(x, edge_index, edge_attr, batch, We1, be1, W1a, b1a, g1, bt1, W1b,
           b1b, We2, be2, W2a, b2a, g2, bt2, W2b, b2b, We3, be3, W3a, b3a, g3,
           bt3, W3b, b3b, Wl1, bl1, Wl2, bl2):
    src = edge_index[0]
    dst = edge_index[1]
    H = W1a.shape[1]
    We2p, We3p = _pad_cols(We2), _pad_cols(We3)
    be2p = jnp.pad(be2, (0, W - H))[None]
    be3p = jnp.pad(be3, (0, W - H))[None]
    W1bp, W2bp = _pad_cols(W1b), _pad_cols(W2b)
    b1bp = jnp.pad(b1b, (0, W - H))[None]
    b2bp = jnp.pad(b2b, (0, W - H))[None]
    W2ap = jnp.pad(W2a, ((0, W - H), (0, 0)))
    W3ap = jnp.pad(W3a, ((0, W - H), (0, 0)))

    e1, e2, e3 = _edge_feats(edge_attr, We1, be1[None], We2p, be2p, We3p, be3p)
    p1 = _sc_aggregate(x, src, dst, e1)
    h1 = _dense(x, p1, W1a, b1a[None], g1[None], bt1[None], W1bp, b1bp)
    p2 = _sc_aggregate(h1, src, dst, e2)
    h2 = _dense(h1, p2, W2ap, b2a[None], g2[None], bt2[None], W2bp, b2bp)
    p3 = _sc_aggregate(h2, src, dst, e3)
    return _dense_final(h2, p3, W3ap, b3a[None], g3[None], bt3[None], W3b,
                        b3b[None], Wl1, bl1[None], Wl2, bl2[None])


# speedup vs baseline: 5.8205x; 1.2653x over previous
import functools

import jax
import jax.numpy as jnp
from jax import lax
from jax.experimental import pallas as pl
from jax.experimental.pallas import tpu as pltpu
from jax.experimental.pallas import tpu_sc as plsc

NC = 2
NS = 16
NW = NC * NS
W = 128
C = 80
CP = C // 8


def _edge1_body(ea_ref, We_ref, be_ref, e_ref):
    e_ref[...] = jnp.dot(ea_ref[...], We_ref[...],
                         preferred_element_type=jnp.float32) + be_ref[...]


def _edge1(ea, We, be):
    E, DE = ea.shape
    BE = 2000
    return pl.pallas_call(
        _edge1_body,
        grid=(E // BE,),
        in_specs=[pl.BlockSpec((BE, DE), lambda i: (i, 0)),
                  pl.BlockSpec((DE, W), lambda i: (0, 0)),
                  pl.BlockSpec((1, W), lambda i: (0, 0))],
        out_specs=pl.BlockSpec((BE, W), lambda i: (i, 0)),
        out_shape=jax.ShapeDtypeStruct((E, W), jnp.float32),
    )(ea, We, be)


def _edge23_body(eag_ref, B2_ref, b2_ref, B3_ref, b3_ref, e2_ref, e3_ref):
    eag = eag_ref[...]
    nch = eag.shape[0] // CP
    t2 = jnp.dot(eag, B2_ref[...],
                 preferred_element_type=jnp.float32) + b2_ref[...]
    e2_ref[...] = t2.reshape(nch, CP, W)
    t3 = jnp.dot(eag, B3_ref[...],
                 preferred_element_type=jnp.float32) + b3_ref[...]
    e3_ref[...] = t3.reshape(nch, CP, W)


def _edge23(eag, B2, b2, B3, b3):
    EG = eag.shape[0]
    BG = 2000
    NCH = BG // CP
    wspec = pl.BlockSpec((W, W), lambda i: (0, 0))
    bspec = pl.BlockSpec((1, W), lambda i: (0, 0))
    ospec = pl.BlockSpec((NCH, CP, W), lambda i: (i, 0, 0))
    oshape = jax.ShapeDtypeStruct((EG // CP, CP, W), jnp.float32)
    return pl.pallas_call(
        _edge23_body,
        grid=(EG // BG,),
        in_specs=[pl.BlockSpec((BG, W), lambda i: (i, 0)),
                  wspec, bspec, wspec, bspec],
        out_specs=[ospec, ospec],
        out_shape=[oshape, oshape],
    )(eag, B2, b2, B3, b3)


def _sc_aggregate(h, src, dst, e, packed):
    N, D = h.shape
    E = src.shape[0]
    CL = C if packed else 40
    EPW = E // NW
    NTOT = EPW // CL
    NB = 3
    NITER = (NTOT + NB) // NB
    ACT = 10
    ROWS = N // ACT
    RZ = 40
    if packed:
        e_slot = pltpu.VMEM((CP, D), jnp.float32)
    else:
        e_slot = pltpu.VMEM((CL, D), jnp.float32)
    mesh = plsc.VectorSubcoreMesh(core_axis_name="c", subcore_axis_name="s")

    @functools.partial(
        pl.kernel,
        out_type=jax.ShapeDtypeStruct((NC, N, D), jnp.float32),
        mesh=mesh,
        scratch_types=[
            [pltpu.VMEM((CL,), jnp.int32)] * NB,
            [pltpu.VMEM((CL,), jnp.int32)] * NB,
            [pltpu.VMEM((CL, D), jnp.float32)] * NB,
            [e_slot] * NB,
            pltpu.VMEM_SHARED((N, D), jnp.float32),
            [pltpu.SemaphoreType.DMA] * NB,
            [pltpu.SemaphoreType.DMA] * NB,
            [pltpu.SemaphoreType.DMA] * NB,
        ],
    )
    def agg(h_hbm, src_hbm, dst_hbm, e_hbm, out_hbm,
            src_v, dst_v, g_v, e_v, aggr_sh, semA, semG, semS):
        cid = lax.axis_index("c")
        sid = lax.axis_index("s")
        wid = cid * NS + sid
        base = wid * EPW

        def fetch(c, b):
            off = base + c * CL
            pltpu.async_copy(src_hbm.at[pl.ds(off, CL)], src_v[b], semA[b])
            pltpu.async_copy(dst_hbm.at[pl.ds(off, CL)], dst_v[b], semA[b])
            if packed:
                pltpu.async_copy(e_hbm.at[wid * NTOT + c], e_v[b], semA[b])
            else:
                pltpu.async_copy(e_hbm.at[pl.ds(off, CL)], e_v[b], semA[b])

        def waitA(b):
            pltpu.make_async_copy(src_hbm.at[pl.ds(0, CL)], src_v[b], semA[b]).wait()
            pltpu.make_async_copy(dst_hbm.at[pl.ds(0, CL)], dst_v[b], semA[b]).wait()
            if packed:
                pltpu.make_async_copy(e_hbm.at[0], e_v[b], semA[b]).wait()
            else:
                pltpu.make_async_copy(e_hbm.at[pl.ds(0, CL)], e_v[b], semA[b]).wait()

        def gather(b):
            pltpu.async_copy(h_hbm.at[src_v[b]], g_v[b], semG[b])

        def waitG(b):
            pltpu.make_async_copy(h_hbm.at[src_v[b]], g_v[b], semG[b]).wait()

        def scatter(b):
            pltpu.async_copy(g_v[b], aggr_sh.at[dst_v[b]], semS[b], add=True)

        def waitS(b):
            pltpu.make_async_copy(g_v[b], aggr_sh.at[dst_v[b]], semS[b]).wait()

        def compute(b):
            if packed:
                @plsc.parallel_loop(0, C)
                def _(i):
                    q = i // 8
                    col = (i % 8) * 16
                    s16 = pl.ds(0, 16)
                    g_v[b][i, s16] = jnp.maximum(
                        g_v[b][i, s16] + e_v[b][q, pl.ds(col, 16)], 0.0)
            else:
                @plsc.parallel_loop(0, CL)
                def _(i):
                    for j in range(D // 16):
                        s = pl.ds(j * 16, 16)
                        g_v[b][i, s] = jnp.maximum(
                            g_v[b][i, s] + e_v[b][i, s], 0.0)

        @plsc.parallel_loop(0, RZ)
        def _(i):
            for j in range(D // 16):
                g_v[0][i, pl.ds(j * 16, 16)] = jnp.zeros((16,), jnp.float32)

        @pl.when(sid < ACT)
        def _():
            for k in range(ROWS // RZ):
                pltpu.sync_copy(g_v[0].at[pl.ds(0, RZ)],
                                aggr_sh.at[pl.ds(sid * ROWS + k * RZ, RZ)])
        plsc.subcore_barrier()

        fetch(0, 0)
        fetch(1, 1)
        waitA(0)
        gather(0)

        def body(k, carry):
            for b in range(NB):
                c = k * NB + b
                nb = (b + 1) % NB

                @pl.when(c + 2 < NTOT)
                def _():
                    fetch(c + 2, (b + 2) % NB)

                @pl.when(c + 1 < NTOT)
                def _():
                    waitA(nb)

                    @pl.when(c >= 2)
                    def _():
                        waitS(nb)
                    gather(nb)

                @pl.when(c < NTOT)
                def _():
                    waitG(b)
                    compute(b)
                    scatter(b)
            return carry

        lax.fori_loop(0, NITER, body, 0)
        for b in range(NB):
            waitS(b)
        plsc.subcore_barrier()

        @pl.when(sid < ACT)
        def _():
            pltpu.sync_copy(aggr_sh.at[pl.ds(sid * ROWS, ROWS)],
                            out_hbm.at[cid, pl.ds(sid * ROWS, ROWS)])

    return agg(h, src, dst, e)


def _mlp_bn(h, Wa_ref, ba_ref, g_ref, bt_ref, Wb_ref, bb_ref):
    t = jnp.dot(h, Wa_ref[...], preferred_element_type=jnp.float32) + ba_ref[...]
    mu = jnp.mean(t, axis=0, keepdims=True)
    var = jnp.mean((t - mu) ** 2, axis=0, keepdims=True)
    t = (t - mu) * lax.rsqrt(var + 1e-5) * g_ref[...] + bt_ref[...]
    t = jnp.maximum(t, 0.0)
    t = jnp.dot(t, Wb_ref[...], preferred_element_type=jnp.float32) + bb_ref[...]
    return jnp.maximum(t, 0.0)


def _dense_body(x_ref, p_ref, Wa_ref, ba_ref, g_ref, bt_ref, Wb_ref, bb_ref,
                o_ref):
    h = x_ref[...] + p_ref[0] + p_ref[1]
    o_ref[...] = _mlp_bn(h, Wa_ref, ba_ref, g_ref, bt_ref, Wb_ref, bb_ref)


def _dense(x, p, Wa, ba, g, bt, Wb, bb):
    N = x.shape[0]
    return pl.pallas_call(
        _dense_body,
        out_shape=jax.ShapeDtypeStruct((N, Wb.shape[1]), jnp.float32),
    )(x, p, Wa, ba, g, bt, Wb, bb)


def _dense_final_body(x_ref, p_ref, Wa_ref, ba_ref, g_ref, bt_ref, Wb_ref,
                      bb_ref, Wl1_ref, bl1_ref, Wl2_ref, bl2_ref, o_ref):
    h = x_ref[...] + p_ref[0] + p_ref[1]
    t = _mlp_bn(h, Wa_ref, ba_ref, g_ref, bt_ref, Wb_ref, bb_ref)
    t = jnp.maximum(
        jnp.dot(t, Wl1_ref[...], preferred_element_type=jnp.float32)
        + bl1_ref[...], 0.0)
    o_ref[...] = jnp.dot(t, Wl2_ref[...],
                         preferred_element_type=jnp.float32) + bl2_ref[...]


def _dense_final(x, p, Wa, ba, g, bt, Wb, bb, Wl1, bl1, Wl2, bl2):
    N = x.shape[0]
    return pl.pallas_call(
        _dense_final_body,
        out_shape=jax.ShapeDtypeStruct((N, Wl2.shape[1]), jnp.float32),
    )(x, p, Wa, ba, g, bt, Wb, bb, Wl1, bl1, Wl2, bl2)


def _pad_cols(a, w=W):
    return jnp.pad(a, ((0, 0), (0, w - a.shape[1])))


def kernel(x, edge_index, edge_attr, batch, We1, be1, W1a, b1a, g1, bt1, W1b,
           b1b, We2, be2, W2a, b2a, g2, bt2, W2b, b2b, We3, be3, W3a, b3a, g3,
           bt3, W3b, b3b, Wl1, bl1, Wl2, bl2):
    src = edge_index[0]
    dst = edge_index[1]
    E = src.shape[0]
    H = W1a.shape[1]
    eye8 = jnp.eye(8, dtype=jnp.float32)
    B2 = jnp.kron(eye8, We2)
    b2 = jnp.tile(be2, 8)[None]
    B3 = jnp.kron(eye8, We3)
    b3 = jnp.tile(be3, 8)[None]
    eag = edge_attr.reshape(E // 8, W)
    W1bp, W2bp = _pad_cols(W1b), _pad_cols(W2b)
    b1bp = jnp.pad(b1b, (0, W - H))[None]
    b2bp = jnp.pad(b2b, (0, W - H))[None]
    W2ap = jnp.pad(W2a, ((0, W - H), (0, 0)))
    W3ap = jnp.pad(W3a, ((0, W - H), (0, 0)))

    e1 = _edge1(edge_attr, We1, be1[None])
    e2, e3 = _edge23(eag, B2, b2, B3, b3)
    p1 = _sc_aggregate(x, src, dst, e1, packed=False)
    h1 = _dense(x, p1, W1a, b1a[None], g1[None], bt1[None], W1bp, b1bp)
    p2 = _sc_aggregate(h1, src, dst, e2, packed=True)
    h2 = _dense(h1, p2, W2ap, b2a[None], g2[None], bt2[None], W2bp, b2bp)
    p3 = _sc_aggregate(h2, src, dst, e3, packed=True)
    return _dense_final(h2, p3, W3ap, b3a[None], g3[None], bt3[None], W3b,
                        b3b[None], Wl1, bl1[None], Wl2, bl2[None])


# speedup vs baseline: 5.8855x; 1.0112x over previous
import functools

import jax
import jax.numpy as jnp
from jax import lax
from jax.experimental import pallas as pl
from jax.experimental.pallas import tpu as pltpu
from jax.experimental.pallas import tpu_sc as plsc

NC = 2
NS = 16
NW = NC * NS
W = 128
C = 80
CP = C // 8


def _rtne16(x):
    u = lax.bitcast_convert_type(x, jnp.uint32)
    return (u + jnp.uint32(0x7FFF) + ((u >> 16) & jnp.uint32(1))) >> 16


def _edge1_body(ea2_ref, Wp_ref, bp_ref, e_ref):
    t = jnp.dot(ea2_ref[...], Wp_ref[...],
                preferred_element_type=jnp.float32) + bp_ref[...]

    def pack(lo, hi):
        return lax.bitcast_convert_type(
            _rtne16(lo) | (_rtne16(hi) << 16), jnp.int32)

    w = jnp.concatenate(
        [pack(t[:, 0:64], t[:, 64:128]), pack(t[:, 128:192], t[:, 192:256])],
        axis=1)
    e_ref[...] = w.reshape(e_ref.shape)


def _edge1(ea2, Wp, bp):
    E2 = ea2.shape[0]
    BE2 = 1000
    return pl.pallas_call(
        _edge1_body,
        grid=(E2 // BE2,),
        in_specs=[pl.BlockSpec((BE2, 32), lambda i: (i, 0)),
                  pl.BlockSpec((32, 2 * W), lambda i: (0, 0)),
                  pl.BlockSpec((1, 2 * W), lambda i: (0, 0))],
        out_specs=pl.BlockSpec((BE2 // 40, 40, W), lambda i: (i, 0, 0)),
        out_shape=jax.ShapeDtypeStruct((E2 // 40, 40, W), jnp.int32),
    )(ea2, Wp, bp)


def _edge23_body(eag_ref, B2_ref, b2_ref, B3_ref, b3_ref, e2_ref, e3_ref):
    eag = eag_ref[...]
    nch = eag.shape[0] // CP
    t2 = jnp.dot(eag, B2_ref[...],
                 preferred_element_type=jnp.float32) + b2_ref[...]
    e2_ref[...] = t2.reshape(nch, CP, W)
    t3 = jnp.dot(eag, B3_ref[...],
                 preferred_element_type=jnp.float32) + b3_ref[...]
    e3_ref[...] = t3.reshape(nch, CP, W)


def _edge23(eag, B2, b2, B3, b3):
    EG = eag.shape[0]
    BG = 2000
    NCH = BG // CP
    wspec = pl.BlockSpec((W, W), lambda i: (0, 0))
    bspec = pl.BlockSpec((1, W), lambda i: (0, 0))
    ospec = pl.BlockSpec((NCH, CP, W), lambda i: (i, 0, 0))
    oshape = jax.ShapeDtypeStruct((EG // CP, CP, W), jnp.float32)
    return pl.pallas_call(
        _edge23_body,
        grid=(EG // BG,),
        in_specs=[pl.BlockSpec((BG, W), lambda i: (i, 0)),
                  wspec, bspec, wspec, bspec],
        out_specs=[ospec, ospec],
        out_shape=[oshape, oshape],
    )(eag, B2, b2, B3, b3)


def _sc_aggregate(h, src, dst, e, packed):
    N, D = h.shape
    E = src.shape[0]
    CL = C
    EPW = E // NW
    NTOT = EPW // CL
    NB = 3
    NITER = (NTOT + NB) // NB
    ACT = 10
    ROWS = N // ACT
    RZ = 40
    if packed:
        e_slot = pltpu.VMEM((CP, D), jnp.float32)
    else:
        e_slot = pltpu.VMEM((CL // 2, D), jnp.int32)
    mesh = plsc.VectorSubcoreMesh(core_axis_name="c", subcore_axis_name="s")

    @functools.partial(
        pl.kernel,
        out_type=jax.ShapeDtypeStruct((NC, N, D), jnp.float32),
        mesh=mesh,
        scratch_types=[
            [pltpu.VMEM((CL,), jnp.int32)] * NB,
            [pltpu.VMEM((CL,), jnp.int32)] * NB,
            [pltpu.VMEM((CL, D), jnp.float32)] * NB,
            [e_slot] * NB,
            pltpu.VMEM_SHARED((N, D), jnp.float32),
            [pltpu.SemaphoreType.DMA] * NB,
            [pltpu.SemaphoreType.DMA] * NB,
            [pltpu.SemaphoreType.DMA] * NB,
        ],
    )
    def agg(h_hbm, src_hbm, dst_hbm, e_hbm, out_hbm,
            src_v, dst_v, g_v, e_v, aggr_sh, semA, semG, semS):
        cid = lax.axis_index("c")
        sid = lax.axis_index("s")
        wid = cid * NS + sid
        base = wid * EPW

        def fetch(c, b):
            off = base + c * CL
            pltpu.async_copy(src_hbm.at[pl.ds(off, CL)], src_v[b], semA[b])
            pltpu.async_copy(dst_hbm.at[pl.ds(off, CL)], dst_v[b], semA[b])
            if packed:
                pltpu.async_copy(e_hbm.at[wid * NTOT + c], e_v[b], semA[b])
            else:
                pltpu.async_copy(e_hbm.at[wid * NTOT + c], e_v[b], semA[b])

        def waitA(b):
            pltpu.make_async_copy(src_hbm.at[pl.ds(0, CL)], src_v[b], semA[b]).wait()
            pltpu.make_async_copy(dst_hbm.at[pl.ds(0, CL)], dst_v[b], semA[b]).wait()
            pltpu.make_async_copy(e_hbm.at[0], e_v[b], semA[b]).wait()

        def gather(b):
            pltpu.async_copy(h_hbm.at[src_v[b]], g_v[b], semG[b])

        def waitG(b):
            pltpu.make_async_copy(h_hbm.at[src_v[b]], g_v[b], semG[b]).wait()

        def scatter(b):
            pltpu.async_copy(g_v[b], aggr_sh.at[dst_v[b]], semS[b], add=True)

        def waitS(b):
            pltpu.make_async_copy(g_v[b], aggr_sh.at[dst_v[b]], semS[b]).wait()

        def compute(b):
            if packed:
                @plsc.parallel_loop(0, C)
                def _(i):
                    q = i // 8
                    col = (i % 8) * 16
                    s16 = pl.ds(0, 16)
                    g_v[b][i, s16] = jnp.maximum(
                        g_v[b][i, s16] + e_v[b][q, pl.ds(col, 16)], 0.0)
            else:
                @plsc.parallel_loop(0, CL // 2)
                def _(q):
                    for half in range(2):
                        r = 2 * q + half
                        for j in range(D // 32):
                            w = e_v[b][q, pl.ds(half * 64 + j * 16, 16)]
                            lo = lax.bitcast_convert_type(
                                w << 16, jnp.float32)
                            hi = lax.bitcast_convert_type(
                                w & jnp.int32(-65536), jnp.float32)
                            sl = pl.ds(j * 16, 16)
                            sh = pl.ds(64 + j * 16, 16)
                            g_v[b][r, sl] = jnp.maximum(g_v[b][r, sl] + lo, 0.0)
                            g_v[b][r, sh] = jnp.maximum(g_v[b][r, sh] + hi, 0.0)

        @plsc.parallel_loop(0, RZ)
        def _(i):
            for j in range(D // 16):
                g_v[0][i, pl.ds(j * 16, 16)] = jnp.zeros((16,), jnp.float32)

        @pl.when(sid < ACT)
        def _():
            for k in range(ROWS // RZ):
                pltpu.sync_copy(g_v[0].at[pl.ds(0, RZ)],
                                aggr_sh.at[pl.ds(sid * ROWS + k * RZ, RZ)])
        plsc.subcore_barrier()

        fetch(0, 0)
        fetch(1, 1)
        waitA(0)
        gather(0)

        def body(k, carry):
            for b in range(NB):
                c = k * NB + b
                nb = (b + 1) % NB

                @pl.when(c + 2 < NTOT)
                def _():
                    fetch(c + 2, (b + 2) % NB)

                @pl.when(c + 1 < NTOT)
                def _():
                    waitA(nb)

                    @pl.when(c >= 2)
                    def _():
                        waitS(nb)
                    gather(nb)

                @pl.when(c < NTOT)
                def _():
                    waitG(b)
                    compute(b)
                    scatter(b)
            return carry

        lax.fori_loop(0, NITER, body, 0)
        for b in range(NB):
            waitS(b)
        plsc.subcore_barrier()

        @pl.when(sid < ACT)
        def _():
            pltpu.sync_copy(aggr_sh.at[pl.ds(sid * ROWS, ROWS)],
                            out_hbm.at[cid, pl.ds(sid * ROWS, ROWS)])

    return agg(h, src, dst, e)


def _mlp_bn(h, Wa_ref, ba_ref, g_ref, bt_ref, Wb_ref, bb_ref):
    t = jnp.dot(h, Wa_ref[...], preferred_element_type=jnp.float32) + ba_ref[...]
    mu = jnp.mean(t, axis=0, keepdims=True)
    var = jnp.mean((t - mu) ** 2, axis=0, keepdims=True)
    t = (t - mu) * lax.rsqrt(var + 1e-5) * g_ref[...] + bt_ref[...]
    t = jnp.maximum(t, 0.0)
    t = jnp.dot(t, Wb_ref[...], preferred_element_type=jnp.float32) + bb_ref[...]
    return jnp.maximum(t, 0.0)


def _dense_body(x_ref, p_ref, Wa_ref, ba_ref, g_ref, bt_ref, Wb_ref, bb_ref,
                o_ref):
    h = x_ref[...] + p_ref[0] + p_ref[1]
    o_ref[...] = _mlp_bn(h, Wa_ref, ba_ref, g_ref, bt_ref, Wb_ref, bb_ref)


def _dense(x, p, Wa, ba, g, bt, Wb, bb):
    N = x.shape[0]
    return pl.pallas_call(
        _dense_body,
        out_shape=jax.ShapeDtypeStruct((N, Wb.shape[1]), jnp.float32),
    )(x, p, Wa, ba, g, bt, Wb, bb)


def _dense_final_body(x_ref, p_ref, Wa_ref, ba_ref, g_ref, bt_ref, Wb_ref,
                      bb_ref, Wl1_ref, bl1_ref, Wl2_ref, bl2_ref, o_ref):
    h = x_ref[...] + p_ref[0] + p_ref[1]
    t = _mlp_bn(h, Wa_ref, ba_ref, g_ref, bt_ref, Wb_ref, bb_ref)
    t = jnp.maximum(
        jnp.dot(t, Wl1_ref[...], preferred_element_type=jnp.float32)
        + bl1_ref[...], 0.0)
    o_ref[...] = jnp.dot(t, Wl2_ref[...],
                         preferred_element_type=jnp.float32) + bl2_ref[...]


def _dense_final(x, p, Wa, ba, g, bt, Wb, bb, Wl1, bl1, Wl2, bl2):
    N = x.shape[0]
    return pl.pallas_call(
        _dense_final_body,
        out_shape=jax.ShapeDtypeStruct((N, Wl2.shape[1]), jnp.float32),
    )(x, p, Wa, ba, g, bt, Wb, bb, Wl1, bl1, Wl2, bl2)


def _pad_cols(a, w=W):
    return jnp.pad(a, ((0, 0), (0, w - a.shape[1])))


def kernel(x, edge_index, edge_attr, batch, We1, be1, W1a, b1a, g1, bt1, W1b,
           b1b, We2, be2, W2a, b2a, g2, bt2, W2b, b2b, We3, be3, W3a, b3a, g3,
           bt3, W3b, b3b, Wl1, bl1, Wl2, bl2):
    src = edge_index[0]
    dst = edge_index[1]
    E = src.shape[0]
    H = W1a.shape[1]
    ea2 = edge_attr.reshape(E // 2, 32)
    B1 = jnp.kron(jnp.eye(2, dtype=jnp.float32), We1)
    b1 = jnp.tile(be1, 2)[None]
    eye8 = jnp.eye(8, dtype=jnp.float32)
    B2 = jnp.kron(eye8, We2)
    b2 = jnp.tile(be2, 8)[None]
    B3 = jnp.kron(eye8, We3)
    b3 = jnp.tile(be3, 8)[None]
    eag = edge_attr.reshape(E // 8, W)
    W1bp, W2bp = _pad_cols(W1b), _pad_cols(W2b)
    b1bp = jnp.pad(b1b, (0, W - H))[None]
    b2bp = jnp.pad(b2b, (0, W - H))[None]
    W2ap = jnp.pad(W2a, ((0, W - H), (0, 0)))
    W3ap = jnp.pad(W3a, ((0, W - H), (0, 0)))

    e1 = _edge1(ea2, B1, b1)
    e2, e3 = _edge23(eag, B2, b2, B3, b3)
    p1 = _sc_aggregate(x, src, dst, e1, packed=False)
    h1 = _dense(x, p1, W1a, b1a[None], g1[None], bt1[None], W1bp, b1bp)
    p2 = _sc_aggregate(h1, src, dst, e2, packed=True)
    h2 = _dense(h1, p2, W2ap, b2a[None], g2[None], bt2[None], W2bp, b2bp)
    p3 = _sc_aggregate(h2, src, dst, e3, packed=True)
    return _dense_final(h2, p3, W3ap, b3a[None], g3[None], bt3[None], W3b,
                        b3b[None], Wl1, bl1[None], Wl2, bl2[None])
